# baseline (speedup 1.0000x reference)
import functools
import math

import jax
import jax.numpy as jnp
from jax import lax
from jax.experimental import pallas as pl
from jax.experimental.pallas import tpu as pltpu

_B, _T, _D, _H, _CS, _DV = 16, 4096, 128, 128, 64, 64
_TILE = 256
_NT = _T // _TILE
_RPB = 2


def _pos_enc(L, Hd):
    pos = jnp.arange(L, dtype=jnp.float32)[:, None]
    div = jnp.exp(jnp.arange(0, Hd, 2, dtype=jnp.float32) * (-math.log(10000.0) / Hd))
    pe = jnp.zeros((L, Hd), dtype=jnp.float32)
    pe = pe.at[:, 0::2].set(jnp.sin(pos * div))
    pe = pe.at[:, 1::2].set(jnp.cos(pos * div))
    return pe


@functools.lru_cache(maxsize=1)
def _consts():
    n = _T // _CS
    pe = _pos_enc(_T, _H)
    gs = []
    for i in range(2):
        g = jax.random.gumbel(jax.random.key(100 + i), (_B * n, _CS, 2), jnp.float32)
        gs.append(g.reshape(_B, _T, 2))
    g4 = jnp.concatenate(gs, axis=-1)
    return jax.device_put(pe), jax.device_put(g4)


def _body(data_ref, v0_ref, v1_ref, we_ref, be_ref, w4_ref, bf4_ref,
          pe_ref, g4_ref, out_ref, acc_ref):
    bf4 = bf4_ref[...]
    vv = [(jnp.broadcast_to(v0_ref[r, 0:1, :], (_TILE, _DV)),
           jnp.broadcast_to(v1_ref[r, 0:1, :], (_TILE, _DV)))
          for r in range(_RPB)]

    acc_ref[...] = jnp.zeros_like(acc_ref)

    iota_t = lax.broadcasted_iota(jnp.int32, (_TILE, _TILE), 0)
    iota_p = lax.broadcasted_iota(jnp.int32, (_TILE, _TILE), 1)
    tri = (iota_t >= iota_p).astype(jnp.float32)

    c = [jnp.int32(0)] * _RPB
    for k in range(_NT):
        psl = slice(k * _TILE, (k + 1) * _TILE)
        for r in range(_RPB):
            base = r * _T
            sl = slice(base + k * _TILE, base + (k + 1) * _TILE)
            dk = data_ref[sl, :]
            hk = ((jnp.dot(dk, we_ref[...], preferred_element_type=jnp.float32)
                   + be_ref[...][None, :]) + pe_ref[psl, :]) * 8.0
            feat = jnp.concatenate([hk, vv[r][0], vv[r][1]], axis=1)
            lg = jnp.dot(feat, w4_ref[...], preferred_element_type=jnp.float32)
            z = g4_ref[r, sl.start - base:sl.stop - base, :] + (lg + bf4)
            m2 = jnp.logical_and(z[:, 0:1] >= z[:, 1:2],
                                 z[:, 2:3] >= z[:, 3:4])
            mf = m2.astype(jnp.float32)
            cs = jnp.dot(tri, mf, preferred_element_type=jnp.float32)
            relp = cs.astype(jnp.int32) - 1
            onehot = (relp == iota_p).astype(jnp.float32) * mf
            vals = lax.dot_general(onehot, hk, (((0,), (0,)), ((), ())),
                                   preferred_element_type=jnp.float32)
            if k == 0:
                acc_ref[base:base + _TILE] = vals
                c[r] = jnp.sum(mf).astype(jnp.int32)
            else:
                pos = base + c[r]
                cur = acc_ref[pl.ds(pos, _TILE)]
                acc_ref[pl.ds(pos, _TILE)] = cur + vals
                c[r] = c[r] + jnp.sum(mf).astype(jnp.int32)
    out_ref[...] = acc_ref[...]


def kernel(data, value_0, value_1, W_embed, b_embed, W_f, b_f):
    pe, g4 = _consts()
    v0 = value_0.reshape(_B, 1, _DV)
    v1 = value_1.reshape(_B, 1, _DV)
    wh = W_f[:_H, :]
    wv = W_f[_H:, :]
    zv = jnp.zeros_like(wv)
    w4 = jnp.concatenate(
        [jnp.concatenate([wh, wv, zv], axis=0),
         jnp.concatenate([wh, zv, wv], axis=0)], axis=1)
    bf4 = jnp.concatenate([b_f, b_f]).reshape(1, 4)

    grid = (_B // _RPB,)
    out = pl.pallas_call(
        _body,
        grid=grid,
        in_specs=[
            pl.BlockSpec((_RPB * _T, _D), lambda b: (b, 0)),
            pl.BlockSpec((_RPB, 1, _DV), lambda b: (b, 0, 0)),
            pl.BlockSpec((_RPB, 1, _DV), lambda b: (b, 0, 0)),
            pl.BlockSpec((_D, _H), lambda b: (0, 0)),
            pl.BlockSpec((_H,), lambda b: (0,)),
            pl.BlockSpec((_H + 2 * _DV, 4), lambda b: (0, 0)),
            pl.BlockSpec((1, 4), lambda b: (0, 0)),
            pl.BlockSpec((_T, _H), lambda b: (0, 0)),
            pl.BlockSpec((_RPB, _T, 4), lambda b: (b, 0, 0)),
        ],
        out_specs=pl.BlockSpec((_RPB * _T, _H), lambda b: (b, 0)),
        out_shape=jax.ShapeDtypeStruct((_B * _T, _H), jnp.float32),
        scratch_shapes=[pltpu.VMEM((_RPB * _T, _H), jnp.float32)],
    )(data.reshape(_B * _T, _D), v0, v1, W_embed, b_embed, w4, bf4,
      pe, g4)
    return out.reshape(_B, _T, _H)
